# baseline (speedup 1.0000x reference)
import functools

import jax
import jax.numpy as jnp
from jax import lax
from jax.experimental import pallas as pl
from jax.experimental.pallas import tpu as pltpu

B = 16384
EMBED_DIM = 128
NUM_GENRES = 100


_CHUNK = 128


def _make_sc_gather():
    from jax.experimental.pallas import tpu_sc as plsc

    info = plsc.get_sparse_core_info()
    nc, ns = info.num_cores, info.num_subcores
    nw = nc * ns
    b_per_w = B // nw
    n_chunks = b_per_w // _CHUNK

    mesh = plsc.VectorSubcoreMesh(core_axis_name="c", subcore_axis_name="s")

    @functools.partial(
        pl.kernel,
        mesh=mesh,
        out_type=jax.ShapeDtypeStruct((B, EMBED_DIM), jnp.float32),
        scratch_types=[
            pltpu.VMEM((n_chunks, _CHUNK), jnp.int32),
            pltpu.VMEM((b_per_w, EMBED_DIM), jnp.float32),
            pltpu.SemaphoreType.DMA,
        ],
    )
    def gather_kernel(idx_hbm, table_hbm, out_hbm, idx_v, rows_v, sem):
        wid = lax.axis_index("s") * nc + lax.axis_index("c")
        base = wid * b_per_w
        pltpu.sync_copy(idx_hbm.at[wid], idx_v)
        for j in range(n_chunks):
            pltpu.async_copy(
                table_hbm.at[idx_v.at[j]],
                rows_v.at[pl.ds(j * _CHUNK, _CHUNK)],
                sem,
            )
        for j in range(n_chunks):
            pltpu.make_async_copy(
                table_hbm.at[idx_v.at[j]],
                rows_v.at[pl.ds(j * _CHUNK, _CHUNK)],
                sem,
            ).wait()
        pltpu.sync_copy(rows_v, out_hbm.at[pl.ds(base, b_per_w)])

    return gather_kernel


_BN = 1024


def _genre_body(gv_ref, wp_ref, bp_ref, w1_ref, b1_ref, a_ref):
    g = jnp.dot(gv_ref[...], wp_ref[...], preferred_element_type=jnp.float32)
    g = jnp.maximum(g + bp_ref[...], 0.0)
    a_ref[...] = (
        jnp.dot(g, w1_ref[EMBED_DIM:, :], preferred_element_type=jnp.float32)
        + b1_ref[...]
    )


def _genre_call(gv, wp, bp, w1, b1):
    full = lambda shape: pl.BlockSpec(shape, lambda i: (0, 0))
    return pl.pallas_call(
        _genre_body,
        grid=(B // _BN,),
        in_specs=[
            pl.BlockSpec((_BN, NUM_GENRES), lambda i: (i, 0)),
            full(wp.shape),
            full(bp.shape),
            full(w1.shape),
            full(b1.shape),
        ],
        out_specs=pl.BlockSpec((_BN, 64), lambda i: (i, 0)),
        out_shape=jax.ShapeDtypeStruct((B, 64), jnp.float32),
    )(gv, wp, bp, w1, b1)


def _head_body(uv_ref, a_ref, w1_ref, w2_ref, b2_ref, out_ref):
    h = jnp.dot(uv_ref[...], w1_ref[:EMBED_DIM, :],
                preferred_element_type=jnp.float32)
    h = jnp.maximum(h + a_ref[...], 0.0)
    out_ref[...] = (
        jnp.dot(h, w2_ref[...], preferred_element_type=jnp.float32) + b2_ref[...]
    )


def _head_call(uv, a, w1, w2, b2):
    full = lambda shape: pl.BlockSpec(shape, lambda i: (0, 0))
    return pl.pallas_call(
        _head_body,
        grid=(B // _BN,),
        in_specs=[
            pl.BlockSpec((_BN, EMBED_DIM), lambda i: (i, 0)),
            pl.BlockSpec((_BN, 64), lambda i: (i, 0)),
            full(w1.shape),
            full(w2.shape),
            full(b2.shape),
        ],
        out_specs=pl.BlockSpec((_BN, 1), lambda i: (i, 0)),
        out_shape=jax.ShapeDtypeStruct((B, 1), jnp.float32),
    )(uv, a, w1, w2, b2)


@jax.jit
def _run(user_ids, genre_vectors, emb_table, W_proj, b_proj, W1, b1, W2, b2):
    gather = _make_sc_gather()
    idx3d = user_ids.astype(jnp.int32).reshape(-1, B // (32 * _CHUNK), _CHUNK)
    uv = gather(idx3d, emb_table)
    a = _genre_call(
        genre_vectors, W_proj, b_proj.reshape(1, EMBED_DIM), W1,
        b1.reshape(1, 64),
    )
    out = _head_call(uv, a, W1, W2, b2.reshape(1, 1))
    return out[:, 0]


def kernel(user_ids, genre_vectors, emb_table, W_proj, b_proj, W1, b1, W2, b2):
    return _run(user_ids, genre_vectors, emb_table, W_proj, b_proj, W1, b1, W2,
                b2)


# speedup vs baseline: 1.2437x; 1.2437x over previous
import functools

import jax
import jax.numpy as jnp
from jax import lax
from jax.experimental import pallas as pl
from jax.experimental.pallas import tpu as pltpu

B = 16384
EMBED_DIM = 128
NUM_GENRES = 100


_CHUNK = 128


def _make_sc_gather():
    from jax.experimental.pallas import tpu_sc as plsc

    info = plsc.get_sparse_core_info()
    nc, ns = info.num_cores, info.num_subcores
    nw = nc * ns
    b_per_w = B // nw
    n_chunks = b_per_w // _CHUNK

    mesh = plsc.VectorSubcoreMesh(core_axis_name="c", subcore_axis_name="s")

    @functools.partial(
        pl.kernel,
        mesh=mesh,
        out_type=jax.ShapeDtypeStruct((B, EMBED_DIM), jnp.float32),
        scratch_types=[
            pltpu.VMEM((n_chunks, _CHUNK), jnp.int32),
            pltpu.VMEM((b_per_w, EMBED_DIM), jnp.float32),
            pltpu.SemaphoreType.DMA,
            pltpu.SemaphoreType.DMA,
        ],
    )
    def gather_kernel(idx_hbm, table_hbm, out_hbm, idx_v, rows_v, gsem, wsem):
        wid = lax.axis_index("s") * nc + lax.axis_index("c")
        base = wid * b_per_w
        pltpu.sync_copy(idx_hbm.at[wid], idx_v)
        for j in range(n_chunks):
            pltpu.async_copy(
                table_hbm.at[idx_v.at[j]],
                rows_v.at[pl.ds(j * _CHUNK, _CHUNK)],
                gsem,
            )
        for j in range(n_chunks):
            pltpu.make_async_copy(
                table_hbm.at[idx_v.at[j]],
                rows_v.at[pl.ds(j * _CHUNK, _CHUNK)],
                gsem,
            ).wait()
            pltpu.async_copy(
                rows_v.at[pl.ds(j * _CHUNK, _CHUNK)],
                out_hbm.at[pl.ds(base + j * _CHUNK, _CHUNK)],
                wsem,
            )
        for j in range(n_chunks):
            pltpu.make_async_copy(
                rows_v.at[pl.ds(j * _CHUNK, _CHUNK)],
                out_hbm.at[pl.ds(base + j * _CHUNK, _CHUNK)],
                wsem,
            ).wait()

    return gather_kernel


_BN = 1024


def _mlp_body(uv_ref, gv_ref, wp_ref, bp_ref, w1_ref, b1_ref, w2r_ref, b2_ref,
              out_ref):
    g = jnp.dot(gv_ref[...], wp_ref[...], preferred_element_type=jnp.float32)
    g = jnp.maximum(g + bp_ref[...], 0.0)
    h = jnp.dot(uv_ref[...], w1_ref[:EMBED_DIM, :],
                preferred_element_type=jnp.float32)
    h = h + jnp.dot(g, w1_ref[EMBED_DIM:, :], preferred_element_type=jnp.float32)
    h = jnp.maximum(h + b1_ref[...], 0.0)
    out_ref[...] = jnp.sum(h * w2r_ref[...], axis=1) + b2_ref[0, 0]


def _mlp_call(uv, gv, wp, bp, w1, b1, w2r, b2):
    full = lambda shape: pl.BlockSpec(shape, lambda i: (0,) * len(shape))
    return pl.pallas_call(
        _mlp_body,
        grid=(B // _BN,),
        in_specs=[
            pl.BlockSpec((_BN, EMBED_DIM), lambda i: (i, 0)),
            pl.BlockSpec((_BN, NUM_GENRES), lambda i: (i, 0)),
            full(wp.shape),
            full(bp.shape),
            full(w1.shape),
            full(b1.shape),
            full(w2r.shape),
            full(b2.shape),
        ],
        out_specs=pl.BlockSpec((_BN,), lambda i: (i,)),
        out_shape=jax.ShapeDtypeStruct((B,), jnp.float32),
    )(uv, gv, wp, bp, w1, b1, w2r, b2)


@jax.jit
def _run(user_ids, genre_vectors, emb_table, W_proj, b_proj, W1, b1, W2, b2):
    gather = _make_sc_gather()
    idx3d = user_ids.astype(jnp.int32).reshape(-1, B // (32 * _CHUNK), _CHUNK)
    uv = gather(idx3d, emb_table)
    return _mlp_call(
        uv,
        genre_vectors,
        W_proj,
        b_proj.reshape(1, EMBED_DIM),
        W1,
        b1.reshape(1, 64),
        W2.reshape(1, 64),
        b2.reshape(1, 1),
    )


def kernel(user_ids, genre_vectors, emb_table, W_proj, b_proj, W1, b1, W2, b2):
    return _run(user_ids, genre_vectors, emb_table, W_proj, b_proj, W1, b1, W2,
                b2)


# speedup vs baseline: 1.4408x; 1.1585x over previous
import functools

import jax
import jax.numpy as jnp
from jax import lax
from jax.experimental import pallas as pl
from jax.experimental.pallas import tpu as pltpu

B = 16384
EMBED_DIM = 128
NUM_GENRES = 100


_CHUNK = 128


def _make_sc_gather():
    from jax.experimental.pallas import tpu_sc as plsc

    info = plsc.get_sparse_core_info()
    nc, ns = info.num_cores, info.num_subcores
    nw = nc * ns
    b_per_w = B // nw
    n_chunks = b_per_w // _CHUNK

    mesh = plsc.VectorSubcoreMesh(core_axis_name="c", subcore_axis_name="s")

    @functools.partial(
        pl.kernel,
        mesh=mesh,
        out_type=jax.ShapeDtypeStruct((B, EMBED_DIM), jnp.float32),
        scratch_types=[
            pltpu.VMEM((n_chunks, _CHUNK), jnp.int32),
            pltpu.VMEM((b_per_w, EMBED_DIM), jnp.float32),
            pltpu.SemaphoreType.DMA,
            pltpu.SemaphoreType.DMA,
        ],
    )
    def gather_kernel(idx_hbm, table_hbm, out_hbm, idx_v, rows_v, gsem, wsem):
        wid = lax.axis_index("s") * nc + lax.axis_index("c")
        base = wid * b_per_w
        pltpu.sync_copy(idx_hbm.at[wid], idx_v)
        for j in range(n_chunks):
            pltpu.async_copy(
                table_hbm.at[idx_v.at[j]],
                rows_v.at[pl.ds(j * _CHUNK, _CHUNK)],
                gsem,
            )
        for j in range(n_chunks):
            pltpu.make_async_copy(
                table_hbm.at[idx_v.at[j]],
                rows_v.at[pl.ds(j * _CHUNK, _CHUNK)],
                gsem,
            ).wait()
            pltpu.async_copy(
                rows_v.at[pl.ds(j * _CHUNK, _CHUNK)],
                out_hbm.at[pl.ds(base + j * _CHUNK, _CHUNK)],
                wsem,
            )
        for j in range(n_chunks):
            pltpu.make_async_copy(
                rows_v.at[pl.ds(j * _CHUNK, _CHUNK)],
                out_hbm.at[pl.ds(base + j * _CHUNK, _CHUNK)],
                wsem,
            ).wait()

    return gather_kernel


_BN = 1024


def _mlp_body(uv_ref, gv_ref, wp_ref, bp_ref, w1_ref, b1_ref, w2r_ref, b2_ref,
              out_ref):
    g = jnp.dot(gv_ref[...], wp_ref[...], preferred_element_type=jnp.float32)
    g = jnp.maximum(g + bp_ref[...], 0.0)
    h = jnp.dot(uv_ref[...], w1_ref[:EMBED_DIM, :],
                preferred_element_type=jnp.float32)
    h = h + jnp.dot(g, w1_ref[EMBED_DIM:, :], preferred_element_type=jnp.float32)
    h = jnp.maximum(h + b1_ref[...], 0.0)
    r = lax.dot_general(w2r_ref[...], h, (((1,), (1,)), ((), ())),
                        preferred_element_type=jnp.float32)
    out_ref[...] = r[0] + b2_ref[0, 0]


def _mlp_call(uv, gv, wp, bp, w1, b1, w2r, b2):
    full = lambda shape: pl.BlockSpec(shape, lambda i: (0,) * len(shape))
    return pl.pallas_call(
        _mlp_body,
        grid=(B // _BN,),
        in_specs=[
            pl.BlockSpec((_BN, EMBED_DIM), lambda i: (i, 0)),
            pl.BlockSpec((_BN, NUM_GENRES), lambda i: (i, 0)),
            full(wp.shape),
            full(bp.shape),
            full(w1.shape),
            full(b1.shape),
            full(w2r.shape),
            full(b2.shape),
        ],
        out_specs=pl.BlockSpec((_BN,), lambda i: (i,)),
        out_shape=jax.ShapeDtypeStruct((B,), jnp.float32),
    )(uv, gv, wp, bp, w1, b1, w2r, b2)


@jax.jit
def _run(user_ids, genre_vectors, emb_table, W_proj, b_proj, W1, b1, W2, b2):
    gather = _make_sc_gather()
    idx3d = user_ids.astype(jnp.int32).reshape(-1, B // (32 * _CHUNK), _CHUNK)
    uv = gather(idx3d, emb_table)
    return _mlp_call(
        uv,
        genre_vectors,
        W_proj,
        b_proj.reshape(1, EMBED_DIM),
        W1,
        b1.reshape(1, 64),
        W2.reshape(1, 64),
        b2.reshape(1, 1),
    )


def kernel(user_ids, genre_vectors, emb_table, W_proj, b_proj, W1, b1, W2, b2):
    return _run(user_ids, genre_vectors, emb_table, W_proj, b_proj, W1, b1, W2,
                b2)


# speedup vs baseline: 1.6450x; 1.1417x over previous
import functools

import jax
import jax.numpy as jnp
from jax import lax
from jax.experimental import pallas as pl
from jax.experimental.pallas import tpu as pltpu

B = 16384
EMBED_DIM = 128
NUM_GENRES = 100


_CHUNK = 128


def _make_sc_gather():
    from jax.experimental.pallas import tpu_sc as plsc

    info = plsc.get_sparse_core_info()
    nc, ns = info.num_cores, info.num_subcores
    nw = nc * ns
    b_per_w = B // nw
    n_chunks = b_per_w // _CHUNK

    mesh = plsc.VectorSubcoreMesh(core_axis_name="c", subcore_axis_name="s")

    @functools.partial(
        pl.kernel,
        mesh=mesh,
        out_type=jax.ShapeDtypeStruct((B, EMBED_DIM), jnp.float32),
        scratch_types=[
            pltpu.VMEM((n_chunks, _CHUNK), jnp.int32),
            pltpu.VMEM((b_per_w, EMBED_DIM), jnp.float32),
            pltpu.SemaphoreType.DMA,
            pltpu.SemaphoreType.DMA,
        ],
    )
    def gather_kernel(idx_hbm, table_hbm, out_hbm, idx_v, rows_v, gsem, wsem):
        wid = lax.axis_index("s") * nc + lax.axis_index("c")
        base = wid * b_per_w
        pltpu.sync_copy(idx_hbm.at[wid], idx_v)
        for j in range(n_chunks):
            pltpu.async_copy(
                table_hbm.at[idx_v.at[j]],
                rows_v.at[pl.ds(j * _CHUNK, _CHUNK)],
                gsem,
            )
        for j in range(n_chunks):
            pltpu.make_async_copy(
                table_hbm.at[idx_v.at[j]],
                rows_v.at[pl.ds(j * _CHUNK, _CHUNK)],
                gsem,
            ).wait()
            pltpu.async_copy(
                rows_v.at[pl.ds(j * _CHUNK, _CHUNK)],
                out_hbm.at[pl.ds(base + j * _CHUNK, _CHUNK)],
                wsem,
            )
        for j in range(n_chunks):
            pltpu.make_async_copy(
                rows_v.at[pl.ds(j * _CHUNK, _CHUNK)],
                out_hbm.at[pl.ds(base + j * _CHUNK, _CHUNK)],
                wsem,
            ).wait()

    return gather_kernel


_BN = 2048


def _mlp_body(uv_ref, gv_ref, wp_ref, bp_ref, w1_ref, b1_ref, w2r_ref, b2_ref,
              out_ref):
    g = jnp.dot(gv_ref[...], wp_ref[...], preferred_element_type=jnp.float32)
    g = jnp.maximum(g + bp_ref[...], 0.0)
    h = jnp.dot(uv_ref[...], w1_ref[:EMBED_DIM, :],
                preferred_element_type=jnp.float32)
    h = h + jnp.dot(g, w1_ref[EMBED_DIM:, :], preferred_element_type=jnp.float32)
    h = jnp.maximum(h + b1_ref[...], 0.0)
    r = lax.dot_general(w2r_ref[...], h, (((1,), (1,)), ((), ())),
                        preferred_element_type=jnp.float32)
    out_ref[...] = r[0] + b2_ref[0, 0]


def _mlp_call(uv, gv, wp, bp, w1, b1, w2r, b2):
    full = lambda shape: pl.BlockSpec(shape, lambda i: (0,) * len(shape))
    return pl.pallas_call(
        _mlp_body,
        grid=(B // _BN,),
        in_specs=[
            pl.BlockSpec((_BN, EMBED_DIM), lambda i: (i, 0)),
            pl.BlockSpec((_BN, NUM_GENRES), lambda i: (i, 0)),
            full(wp.shape),
            full(bp.shape),
            full(w1.shape),
            full(b1.shape),
            full(w2r.shape),
            full(b2.shape),
        ],
        out_specs=pl.BlockSpec((_BN,), lambda i: (i,)),
        out_shape=jax.ShapeDtypeStruct((B,), jnp.float32),
    )(uv, gv, wp, bp, w1, b1, w2r, b2)


@jax.jit
def _run(user_ids, genre_vectors, emb_table, W_proj, b_proj, W1, b1, W2, b2):
    gather = _make_sc_gather()
    idx3d = user_ids.astype(jnp.int32).reshape(-1, B // (32 * _CHUNK), _CHUNK)
    uv = gather(idx3d, emb_table)
    return _mlp_call(
        uv,
        genre_vectors,
        W_proj,
        b_proj.reshape(1, EMBED_DIM),
        W1,
        b1.reshape(1, 64),
        W2.reshape(1, 64),
        b2.reshape(1, 1),
    )


def kernel(user_ids, genre_vectors, emb_table, W_proj, b_proj, W1, b1, W2, b2):
    return _run(user_ids, genre_vectors, emb_table, W_proj, b_proj, W1, b1, W2,
                b2)


# speedup vs baseline: 1.7230x; 1.0474x over previous
import functools

import jax
import jax.numpy as jnp
from jax import lax
from jax.experimental import pallas as pl
from jax.experimental.pallas import tpu as pltpu

B = 16384
EMBED_DIM = 128
NUM_GENRES = 100


_CHUNK = 128


def _make_sc_gather():
    from jax.experimental.pallas import tpu_sc as plsc

    info = plsc.get_sparse_core_info()
    nc, ns = info.num_cores, info.num_subcores
    nw = nc * ns
    b_per_w = B // nw
    n_chunks = b_per_w // _CHUNK

    mesh = plsc.VectorSubcoreMesh(core_axis_name="c", subcore_axis_name="s")

    @functools.partial(
        pl.kernel,
        mesh=mesh,
        out_type=jax.ShapeDtypeStruct((B, EMBED_DIM), jnp.float32),
        scratch_types=[
            pltpu.VMEM((n_chunks, _CHUNK), jnp.int32),
            pltpu.VMEM((b_per_w, EMBED_DIM), jnp.float32),
            pltpu.SemaphoreType.DMA,
            pltpu.SemaphoreType.DMA,
        ],
    )
    def gather_kernel(idx_hbm, table_hbm, out_hbm, idx_v, rows_v, gsem, wsem):
        wid = lax.axis_index("s") * nc + lax.axis_index("c")
        base = wid * b_per_w
        pltpu.sync_copy(idx_hbm.at[wid], idx_v)
        for j in range(n_chunks):
            pltpu.async_copy(
                table_hbm.at[idx_v.at[j]],
                rows_v.at[pl.ds(j * _CHUNK, _CHUNK)],
                gsem,
            )
        for j in range(n_chunks):
            pltpu.make_async_copy(
                table_hbm.at[idx_v.at[j]],
                rows_v.at[pl.ds(j * _CHUNK, _CHUNK)],
                gsem,
            ).wait()
            pltpu.async_copy(
                rows_v.at[pl.ds(j * _CHUNK, _CHUNK)],
                out_hbm.at[pl.ds(base + j * _CHUNK, _CHUNK)],
                wsem,
            )
        for j in range(n_chunks):
            pltpu.make_async_copy(
                rows_v.at[pl.ds(j * _CHUNK, _CHUNK)],
                out_hbm.at[pl.ds(base + j * _CHUNK, _CHUNK)],
                wsem,
            ).wait()

    return gather_kernel


_BN = 8192


def _mlp_body(uv_ref, gv_ref, wp_ref, bp_ref, w1_ref, b1_ref, w2r_ref, b2_ref,
              out_ref):
    g = jnp.dot(gv_ref[...], wp_ref[...], preferred_element_type=jnp.float32)
    g = jnp.maximum(g + bp_ref[...], 0.0)
    h = jnp.dot(uv_ref[...], w1_ref[:EMBED_DIM, :],
                preferred_element_type=jnp.float32)
    h = h + jnp.dot(g, w1_ref[EMBED_DIM:, :], preferred_element_type=jnp.float32)
    h = jnp.maximum(h + b1_ref[...], 0.0)
    r = lax.dot_general(w2r_ref[...], h, (((1,), (1,)), ((), ())),
                        preferred_element_type=jnp.float32)
    out_ref[...] = r[0] + b2_ref[0, 0]


def _mlp_call(uv, gv, wp, bp, w1, b1, w2r, b2):
    full = lambda shape: pl.BlockSpec(shape, lambda i: (0,) * len(shape))
    return pl.pallas_call(
        _mlp_body,
        grid=(B // _BN,),
        in_specs=[
            pl.BlockSpec((_BN, EMBED_DIM), lambda i: (i, 0)),
            pl.BlockSpec((_BN, NUM_GENRES), lambda i: (i, 0)),
            full(wp.shape),
            full(bp.shape),
            full(w1.shape),
            full(b1.shape),
            full(w2r.shape),
            full(b2.shape),
        ],
        out_specs=pl.BlockSpec((_BN,), lambda i: (i,)),
        out_shape=jax.ShapeDtypeStruct((B,), jnp.float32),
    )(uv, gv, wp, bp, w1, b1, w2r, b2)


@jax.jit
def _run(user_ids, genre_vectors, emb_table, W_proj, b_proj, W1, b1, W2, b2):
    gather = _make_sc_gather()
    idx3d = user_ids.astype(jnp.int32).reshape(-1, B // (32 * _CHUNK), _CHUNK)
    uv = gather(idx3d, emb_table)
    return _mlp_call(
        uv,
        genre_vectors,
        W_proj,
        b_proj.reshape(1, EMBED_DIM),
        W1,
        b1.reshape(1, 64),
        W2.reshape(1, 64),
        b2.reshape(1, 1),
    )


def kernel(user_ids, genre_vectors, emb_table, W_proj, b_proj, W1, b1, W2, b2):
    return _run(user_ids, genre_vectors, emb_table, W_proj, b_proj, W1, b1, W2,
                b2)


# speedup vs baseline: 1.8466x; 1.0718x over previous
import functools

import jax
import jax.numpy as jnp
from jax import lax
from jax.experimental import pallas as pl
from jax.experimental.pallas import tpu as pltpu

B = 16384
EMBED_DIM = 128
NUM_GENRES = 100


_CHUNK = 128


def _make_sc_gather():
    from jax.experimental.pallas import tpu_sc as plsc

    info = plsc.get_sparse_core_info()
    nc, ns = info.num_cores, info.num_subcores
    nw = nc * ns
    b_per_w = B // nw
    n_chunks = b_per_w // _CHUNK

    mesh = plsc.VectorSubcoreMesh(core_axis_name="c", subcore_axis_name="s")

    @functools.partial(
        pl.kernel,
        mesh=mesh,
        out_type=jax.ShapeDtypeStruct((B, EMBED_DIM), jnp.float32),
        scratch_types=[
            pltpu.VMEM((n_chunks, _CHUNK), jnp.int32),
            pltpu.VMEM((b_per_w, EMBED_DIM), jnp.float32),
            pltpu.SemaphoreType.DMA,
            pltpu.SemaphoreType.DMA,
        ],
    )
    def gather_kernel(idx_hbm, table_hbm, out_hbm, idx_v, rows_v, gsem, wsem):
        wid = lax.axis_index("s") * nc + lax.axis_index("c")
        base = wid * b_per_w
        pltpu.sync_copy(idx_hbm.at[wid], idx_v)
        for j in range(n_chunks):
            pltpu.async_copy(
                table_hbm.at[idx_v.at[j]],
                rows_v.at[pl.ds(j * _CHUNK, _CHUNK)],
                gsem,
            )
        for j in range(n_chunks):
            pltpu.make_async_copy(
                table_hbm.at[idx_v.at[j]],
                rows_v.at[pl.ds(j * _CHUNK, _CHUNK)],
                gsem,
            ).wait()
            pltpu.async_copy(
                rows_v.at[pl.ds(j * _CHUNK, _CHUNK)],
                out_hbm.at[pl.ds(base + j * _CHUNK, _CHUNK)],
                wsem,
            )
        for j in range(n_chunks):
            pltpu.make_async_copy(
                rows_v.at[pl.ds(j * _CHUNK, _CHUNK)],
                out_hbm.at[pl.ds(base + j * _CHUNK, _CHUNK)],
                wsem,
            ).wait()

    return gather_kernel


_BN = 8192


def _mlp_body(uv_ref, gv_ref, wp_ref, bp_ref, w1_ref, b1_ref, w2r_ref, b2_ref,
              out_ref):
    g = jnp.dot(gv_ref[...], wp_ref[...], preferred_element_type=jnp.float32)
    g = jnp.maximum(g + bp_ref[...], 0.0)
    h = jnp.dot(uv_ref[...], w1_ref[:EMBED_DIM, :],
                preferred_element_type=jnp.float32)
    h = h + jnp.dot(g, w1_ref[EMBED_DIM:, :], preferred_element_type=jnp.float32)
    h = jnp.maximum(h + b1_ref[...], 0.0)
    r = lax.dot_general(w2r_ref[...], h, (((1,), (1,)), ((), ())),
                        preferred_element_type=jnp.float32)
    out_ref[...] = r[0] + b2_ref[0, 0]


def _mlp_call(uv, gv, wp, bp, w1, b1, w2r, b2):
    full = lambda shape: pl.BlockSpec(shape, lambda i: (0,) * len(shape))
    return pl.pallas_call(
        _mlp_body,
        grid=(B // _BN,),
        in_specs=[
            pl.BlockSpec((_BN, EMBED_DIM), lambda i: (i, 0)),
            pl.BlockSpec((_BN, NUM_GENRES), lambda i: (i, 0)),
            full(wp.shape),
            full(bp.shape),
            full(w1.shape),
            full(b1.shape),
            full(w2r.shape),
            full(b2.shape),
        ],
        out_specs=pl.BlockSpec((_BN,), lambda i: (i,)),
        out_shape=jax.ShapeDtypeStruct((B,), jnp.float32),
    )(uv, gv, wp, bp, w1, b1, w2r, b2)


@jax.jit
def _run(user_ids, genre_vectors, emb_table, W_proj, b_proj, W1, b1, W2, b2):
    gather = _make_sc_gather()
    idx3d = user_ids.astype(jnp.int32).reshape(-1, B // (32 * _CHUNK), _CHUNK)
    uv = gather(idx3d, emb_table)
    return _mlp_call(
        uv,
        genre_vectors.astype(jnp.bfloat16),
        W_proj.astype(jnp.bfloat16),
        b_proj.reshape(1, EMBED_DIM),
        W1,
        b1.reshape(1, 64),
        W2.reshape(1, 64),
        b2.reshape(1, 1),
    )


def kernel(user_ids, genre_vectors, emb_table, W_proj, b_proj, W1, b1, W2, b2):
    return _run(user_ids, genre_vectors, emb_table, W_proj, b_proj, W1, b1, W2,
                b2)
